# Initial kernel scaffold; baseline (speedup 1.0000x reference)
import jax
import jax.numpy as jnp
from jax.experimental import pallas as pl


def kernel(keys, dnd_keys, dnd_values):
    raise NotImplementedError("write your pallas kernel here")


# speedup vs baseline: 5.1936x; 5.1936x over previous
import jax
import jax.numpy as jnp
from jax.experimental import pallas as pl
from jax.experimental.pallas import tpu as pltpu

_CAP = 100000
_PAD_CAP = 102400
_K = 64
_Q = 1024
_QB = 64
_NCHUNK = 8
_CB = _PAD_CAP // _NCHUNK
_KNN = 50
_DELTA = 0.001
_INF_BITS = 0x7F800000
_BSEARCH_ITERS = 31


def _dnd_kernel(q_ref, kt_ref, v_ref, out_ref, dist_ref):
    c = pl.program_id(1)

    q = q_ref[...]
    kt = kt_ref[0]
    qsq = jnp.sum(q * q, axis=1, keepdims=True)
    dsq = jnp.sum(kt * kt, axis=0, keepdims=True)
    prod = jax.lax.dot_general(
        q, kt, (((1,), (0,)), ((), ())),
        preferred_element_type=jnp.float32)
    dist = jnp.maximum(qsq + dsq - 2.0 * prod, 0.0)
    dist_ref[c] = dist

    @pl.when(c == _NCHUNK - 1)
    def _():
        zero = jnp.zeros((_QB, 1), dtype=jnp.float32)

        def count_le(t):
            def cbody(k, acc):
                m = (dist_ref[k] <= t).astype(jnp.float32)
                return acc + jnp.sum(m, axis=1, keepdims=True)
            return jax.lax.fori_loop(0, _NCHUNK, cbody, zero)

        def body(_, carry):
            lo, hi = carry
            mid = lo + ((hi - lo) >> 1)
            t = jax.lax.bitcast_convert_type(mid, jnp.float32)
            ge = count_le(t) >= float(_KNN)
            hi = jnp.where(ge, mid, hi)
            lo = jnp.where(ge, lo, mid)
            return lo, hi

        lo0 = jnp.full((_QB, 1), -1, dtype=jnp.int32)
        hi0 = jnp.full((_QB, 1), _INF_BITS, dtype=jnp.int32)
        _, hi = jax.lax.fori_loop(0, _BSEARCH_ITERS, body, (lo0, hi0))

        t50 = jax.lax.bitcast_convert_type(hi, jnp.float32)

        def tally(k, carry):
            n_lt, n_eq = carry
            dk = dist_ref[k]
            n_lt = n_lt + jnp.sum((dk < t50).astype(jnp.float32),
                                  axis=1, keepdims=True)
            n_eq = n_eq + jnp.sum((dk == t50).astype(jnp.float32),
                                  axis=1, keepdims=True)
            return n_lt, n_eq

        n_lt, n_eq = jax.lax.fori_loop(0, _NCHUNK, tally, (zero, zero))
        frac = (float(_KNN) - n_lt) / n_eq

        def wsum(k, carry):
            num, den = carry
            dk = dist_ref[k]
            coef = jnp.where(dk < t50, 1.0, 0.0) + frac * (dk == t50)
            r = coef / (dk + _DELTA)
            num = num + jnp.sum(r * v_ref[k], axis=1, keepdims=True)
            den = den + jnp.sum(r, axis=1, keepdims=True)
            return num, den

        num, den = jax.lax.fori_loop(0, _NCHUNK, wsum, (zero, zero))
        out_ref[...] = num / den


def kernel(keys, dnd_keys, dnd_values):
    kt = jnp.concatenate(
        [dnd_keys.T,
         jnp.full((_K, _PAD_CAP - _CAP), 1e6, dtype=jnp.float32)], axis=1)
    kt = kt.reshape(_K, _NCHUNK, _CB).transpose(1, 0, 2)
    vals = jnp.concatenate(
        [dnd_values, jnp.zeros((_PAD_CAP - _CAP,), dtype=jnp.float32)])
    vals = vals.reshape(_NCHUNK, 1, _CB)

    return pl.pallas_call(
        _dnd_kernel,
        grid=(_Q // _QB, _NCHUNK),
        in_specs=[
            pl.BlockSpec((_QB, _K), lambda i, c: (i, 0)),
            pl.BlockSpec((1, _K, _CB), lambda i, c: (c, 0, 0)),
            pl.BlockSpec((_NCHUNK, 1, _CB), lambda i, c: (0, 0, 0)),
        ],
        out_specs=pl.BlockSpec((_QB, 1), lambda i, c: (i, 0)),
        out_shape=jax.ShapeDtypeStruct((_Q, 1), jnp.float32),
        scratch_shapes=[pltpu.VMEM((_NCHUNK, _QB, _CB), jnp.float32)],
    )(keys, kt, vals)
